# baseline (speedup 1.0000x reference)
import functools
import jax
import jax.numpy as jnp
from jax import lax
from jax.experimental import pallas as pl
from jax.experimental.pallas import tpu as pltpu
from jax.experimental.pallas import tpu_sc as plsc

D_M = 64
PAD_W = 128
OPAD = 208
SCALE = 8.0
NW = 32
LANES = 16
NBUF = 4
AHEAD = 2
SLAB = 16


def _gather_call(R, C):
    RW = R // NW
    NS = RW // SLAB
    NG = C // LANES
    NT = C - NG * LANES
    mesh = plsc.VectorSubcoreMesh(core_axis_name="c", subcore_axis_name="s")

    @functools.partial(
        pl.kernel,
        mesh=mesh,
        out_type=jax.ShapeDtypeStruct((R, C, PAD_W), jnp.float32),
        compiler_params=pltpu.CompilerParams(use_tc_tiling_on_sc=False),
        scratch_types=[
            pltpu.VMEM((2, SLAB, C), jnp.int32),
            pltpu.VMEM((2, SLAB, OPAD), jnp.int32),
            pltpu.VMEM((NBUF, C, PAD_W), jnp.float32),
            pltpu.SemaphoreType.DMA((2,)),
            pltpu.SemaphoreType.DMA((2,)),
            pltpu.SemaphoreType.DMA((NBUF,)),
            pltpu.SemaphoreType.DMA((NBUF,)),
        ],
    )
    def body(xh_hbm, xo_hbm, lutd_hbm, out_hbm, ih, io, bufs,
             hsems, osems_i, gsems, osems):
        wid = lax.axis_index("s") * 2 + lax.axis_index("c")
        rbase = wid * RW

        def stage(s, sem_wait=False):
            rows = pl.ds(rbase + s * SLAB, SLAB)
            argsH = (xh_hbm.at[rows], ih.at[s % 2], hsems.at[s % 2])
            argsO = (xo_hbm.at[rows], io.at[s % 2], osems_i.at[s % 2])
            if sem_wait:
                pltpu.make_async_copy(*argsH).wait()
                pltpu.make_async_copy(*argsO).wait()
            else:
                pltpu.async_copy(*argsH)
                pltpu.async_copy(*argsO)

        def slab_row(j):
            return (j // SLAB) % 2, j % SLAB

        def gather(j, b, start):
            sb, rs = slab_row(j)
            args = (lutd_hbm.at[ih.at[sb, rs]], bufs.at[b], gsems.at[b])
            if start:
                pltpu.async_copy(*args)
            else:
                pltpu.make_async_copy(*args).wait()

        def store(j, b, start):
            args = (
                bufs.at[b, :, pl.ds(0, D_M)],
                out_hbm.at[rbase + j, :, pl.ds(0, D_M)],
                osems.at[b],
            )
            if start:
                pltpu.async_copy(*args)
            else:
                pltpu.make_async_copy(*args).wait()

        def compact_group(b, sb, rs, g, nrows):
            par = io[sb, rs, pl.ds(g * LANES, LANES)]
            for i in range(nrows):
                r = g * LANES + i
                off = par[i] * D_M
                for q in range(D_M // LANES):
                    bufs[b, r, pl.ds(q * LANES, LANES)] = (
                        bufs[b, r, pl.ds(off + q * LANES, LANES)] * SCALE
                    )

        stage(0)
        stage(1)
        stage(0, sem_wait=True)
        for b in range(AHEAD):
            gather(b, b, start=True)

        def slab_loop(s, carry):
            @pl.when(s + 1 < NS)
            def _wait_next():
                stage(s + 1, sem_wait=True)

            def block(j0, c1):
                for b in range(NBUF):
                    j = j0 + b
                    jf = j + AHEAD
                    bf = (b + AHEAD) % NBUF

                    @pl.when(jf < RW)
                    def _fire():
                        @pl.when(jf >= NBUF)
                        def _drain():
                            store(jf - NBUF, bf, start=False)

                        gather(jf, bf, start=True)

                    gather(j, b, start=False)

                    sb, rs = slab_row(j)

                    def grp(g, c2):
                        compact_group(b, sb, rs, g, LANES)
                        return c2

                    lax.fori_loop(0, NG, grp, 0)
                    compact_group(b, sb, rs, NG, NT)

                    store(j, b, start=True)
                return c1

            lax.fori_loop(
                0, SLAB // NBUF, lambda t, c: block(s * SLAB + t * NBUF, c), 0
            )

            @pl.when(s + 2 < NS)
            def _restage():
                stage(s + 2)

            return carry

        lax.fori_loop(0, NS, slab_loop, 0)

        for b in range(NBUF):
            store(RW - NBUF + b, b, start=False)

    return body


def kernel(x, lut):
    xi = x.astype(jnp.int32)
    xh = xi >> 1
    xo = jnp.pad(xi & 1, ((0, 0), (0, OPAD - x.shape[1])))
    lutd = lut.reshape(lut.shape[0] // 2, 2 * D_M)
    outp = _gather_call(x.shape[0], x.shape[1])(xh, xo, lutd)
    return outp[:, :, :D_M]
